# baseline (speedup 1.0000x reference)
import math

import numpy as np

import jax
import jax.numpy as jnp
from jax import lax
from jax.experimental import pallas as pl
from jax.experimental.pallas import tpu as pltpu
from jax.experimental.pallas import tpu_sc as plsc

D = 64
SCALE = math.sqrt(D)
NC, NS, L = 2, 16, 16
NW = NC * NS
BB = 128
NBUF = 5

_I = np.arange(L, dtype=np.int32)
_DIAG = [np.int32((_I + k) % L) for k in range(L)]


def _body(x_hbm, table_hbm, out_hbm, idx_v, rows, tpose, gsems, osems):
    n_chunks = x_hbm.shape[1]
    wid = lax.axis_index("s") * NC + lax.axis_index("c")
    pltpu.sync_copy(x_hbm.at[wid], idx_v)

    def gather(c, b):
        return pltpu.make_async_copy(
            table_hbm.at[idx_v.at[c]], rows[b], gsems[b])

    def ocopy(c, b):
        return pltpu.make_async_copy(
            tpose[b], out_hbm.at[c, :, wid], osems[b])

    biota = lax.iota(jnp.int32, L)
    landbase = (biota & 7) * BB
    dvecs = [biota + d0 for d0 in range(0, D, L)]
    hvecs = [dv >> 3 for dv in dvecs]

    def transpose_scale(b):
        rows_b = rows[b]
        tp = tpose[b]

        def b0_step(g, _):
            bsp = jnp.full((L,), g * L, jnp.int32)
            for k in range(L):
                rot = (biota + k) & (L - 1)
                bvec = rot + bsp
                lvec = landbase + bvec
                for j in range(D // L):
                    v = plsc.load_gather(rows_b, [bvec, dvecs[j]])
                    plsc.store_scatter(tp, [hvecs[j], lvec], v * SCALE)
            return 0

        lax.fori_loop(0, BB // L, b0_step, 0, unroll=4)

    for b in range(NBUF - 1):
        gather(b, b).start()

    def step(c, b):
        @pl.when(c + NBUF - 1 < n_chunks)
        def _():
            gather(c + NBUF - 1, (b + NBUF - 1) % NBUF).start()

        gather(c, b).wait()

        @pl.when(c >= NBUF)
        def _():
            ocopy(c - NBUF, b).wait()

        transpose_scale(b)
        ocopy(c, b).start()

    def outer(k, _):
        for b in range(NBUF):
            step(k * NBUF + b, b)
        return 0

    lax.fori_loop(0, n_chunks // NBUF, outer, 0)
    for b in range(NBUF):
        ocopy(n_chunks - NBUF + b, b).wait()


@jax.jit
def kernel(x, table):
    B, S = x.shape
    x_in = jnp.transpose(x).reshape(S, NW, BB).transpose(1, 0, 2)

    run = pl.kernel(
        _body,
        out_type=jax.ShapeDtypeStruct((S, D // 8, NW, 8 * BB), jnp.float32),
        mesh=plsc.VectorSubcoreMesh(
            core_axis_name="c", subcore_axis_name="s", num_cores=NC,
            num_subcores=NS),
        scratch_types=[
            pltpu.VMEM((S, BB), jnp.int32),
            [pltpu.VMEM((BB, D), jnp.float32)] * NBUF,
            [pltpu.VMEM((D // 8, 8 * BB), jnp.float32)] * NBUF,
            [pltpu.SemaphoreType.DMA] * NBUF,
            [pltpu.SemaphoreType.DMA] * NBUF,
        ],
        compiler_params=pltpu.CompilerParams(
            use_tc_tiling_on_sc=False, needs_layout_passes=False),
    )
    o5 = run(x_in, table)
    return (o5.reshape(S, D // 8, NW, 8, BB)
            .transpose(2, 4, 0, 1, 3).reshape(B, S, D))


# speedup vs baseline: 1.0209x; 1.0209x over previous
import math

import numpy as np

import jax
import jax.numpy as jnp
from jax import lax
from jax.experimental import pallas as pl
from jax.experimental.pallas import tpu as pltpu
from jax.experimental.pallas import tpu_sc as plsc

D = 64
SCALE = math.sqrt(D)
NC, NS, L = 2, 16, 16
NW = NC * NS
BB = 128
NBUF = 5

_I = np.arange(L, dtype=np.int32)
_DIAG = [np.int32((_I + k) % L) for k in range(L)]


def _body(x_hbm, table_hbm, out_hbm, idx_v, rows, tpose, gsems, osems):
    n_chunks = x_hbm.shape[1]
    wid = lax.axis_index("s") * NC + lax.axis_index("c")
    pltpu.sync_copy(x_hbm.at[wid], idx_v)

    def gather(c, b):
        return pltpu.make_async_copy(
            table_hbm.at[idx_v.at[c]], rows[b], gsems[b])

    def ocopy(c, b):
        return pltpu.make_async_copy(
            tpose[b], out_hbm.at[c, :, wid], osems[b])

    biota = lax.iota(jnp.int32, L)
    landbase = (biota & 7) * BB
    dvecs = [biota + d0 for d0 in range(0, D, L)]
    hvecs = [dv >> 3 for dv in dvecs]

    def transpose_scale(b):
        rows_b = rows[b]
        tp = tpose[b]

        def b_step(bb, _):
            lvec = landbase + bb
            for j in range(D // L):
                v = rows_b[bb, pl.ds(j * L, L)]
                plsc.store_scatter(tp, [hvecs[j], lvec], v * SCALE)
            return 0

        lax.fori_loop(0, BB, b_step, 0, unroll=4)

    for b in range(NBUF - 1):
        gather(b, b).start()

    def step(c, b):
        @pl.when(c + NBUF - 1 < n_chunks)
        def _():
            gather(c + NBUF - 1, (b + NBUF - 1) % NBUF).start()

        gather(c, b).wait()

        @pl.when(c >= NBUF)
        def _():
            ocopy(c - NBUF, b).wait()

        transpose_scale(b)
        ocopy(c, b).start()

    def outer(k, _):
        for b in range(NBUF):
            step(k * NBUF + b, b)
        return 0

    lax.fori_loop(0, n_chunks // NBUF, outer, 0)
    for b in range(NBUF):
        ocopy(n_chunks - NBUF + b, b).wait()


@jax.jit
def kernel(x, table):
    B, S = x.shape
    x_in = jnp.transpose(x).reshape(S, NW, BB).transpose(1, 0, 2)

    run = pl.kernel(
        _body,
        out_type=jax.ShapeDtypeStruct((S, D // 8, NW, 8 * BB), jnp.float32),
        mesh=plsc.VectorSubcoreMesh(
            core_axis_name="c", subcore_axis_name="s", num_cores=NC,
            num_subcores=NS),
        scratch_types=[
            pltpu.VMEM((S, BB), jnp.int32),
            [pltpu.VMEM((BB, D), jnp.float32)] * NBUF,
            [pltpu.VMEM((D // 8, 8 * BB), jnp.float32)] * NBUF,
            [pltpu.SemaphoreType.DMA] * NBUF,
            [pltpu.SemaphoreType.DMA] * NBUF,
        ],
        compiler_params=pltpu.CompilerParams(
            use_tc_tiling_on_sc=False, needs_layout_passes=False),
    )
    o5 = run(x_in, table)
    return (o5.reshape(S, D // 8, NW, 8, BB)
            .transpose(2, 4, 0, 1, 3).reshape(B, S, D))


# speedup vs baseline: 1.5459x; 1.5141x over previous
import math

import jax
import jax.numpy as jnp
from jax import lax
from jax.experimental import pallas as pl
from jax.experimental.pallas import tpu as pltpu
from jax.experimental.pallas import tpu_sc as plsc

D = 64
SCALE = math.sqrt(D)
NC, NS, L = 2, 16, 16
NW = NC * NS
BB = 128
NBUF = 4


def _body(x_hbm, table_hbm, out_hbm, idx_v, rows, tpose, gsems, osems):
    n_chunks = x_hbm.shape[1]
    wid = lax.axis_index("s") * NC + lax.axis_index("c")
    pltpu.sync_copy(x_hbm.at[wid], idx_v)

    def gather(c, b):
        return pltpu.make_async_copy(
            table_hbm.at[idx_v.at[c]], rows[b], gsems[b])

    def ocopy(c, b):
        return pltpu.make_async_copy(
            tpose[b], out_hbm.at[c, :, wid], osems[b])

    biota = lax.iota(jnp.int32, L)
    landbase = (biota & 7) * BB
    dvecs = [biota + d0 for d0 in range(0, D, L)]
    hvecs = [dv >> 3 for dv in dvecs]

    def transpose_scale(b):
        rows_b = rows[b]
        tp = tpose[b]

        def b0_step(g, _):
            bsp = jnp.full((L,), g * L, jnp.int32)
            for k in range(L):
                rot = (biota + k) & (L - 1)
                bvec = rot + bsp
                lvec = landbase + bvec
                for j in range(D // L):
                    v = plsc.load_gather(rows_b, [bvec, dvecs[j]])
                    plsc.store_scatter(tp, [hvecs[j], lvec], v * SCALE)
            return 0

        lax.fori_loop(0, BB // L, b0_step, 0, unroll=2)

    for b in range(NBUF - 1):
        gather(b, b).start()

    def step(c, b):
        @pl.when(c + NBUF - 1 < n_chunks)
        def _():
            gather(c + NBUF - 1, (b + NBUF - 1) % NBUF).start()

        gather(c, b).wait()

        @pl.when(c >= NBUF)
        def _():
            ocopy(c - NBUF, b).wait()

        transpose_scale(b)
        ocopy(c, b).start()

    def outer(k, _):
        for b in range(NBUF):
            step(k * NBUF + b, b)
        return 0

    lax.fori_loop(0, n_chunks // NBUF, outer, 0)
    for b in range(NBUF):
        ocopy(n_chunks - NBUF + b, b).wait()


@jax.jit
def kernel(x, table):
    B, S = x.shape
    x_in = jnp.transpose(x).reshape(S, NW, BB).transpose(1, 0, 2)

    run = pl.kernel(
        _body,
        out_type=jax.ShapeDtypeStruct((S, D // 8, NW, 8 * BB), jnp.float32),
        mesh=plsc.VectorSubcoreMesh(
            core_axis_name="c", subcore_axis_name="s", num_cores=NC,
            num_subcores=NS),
        scratch_types=[
            pltpu.VMEM((S, BB), jnp.int32),
            [pltpu.VMEM((BB, D), jnp.float32)] * NBUF,
            [pltpu.VMEM((D // 8, 8 * BB), jnp.float32)] * NBUF,
            [pltpu.SemaphoreType.DMA] * NBUF,
            [pltpu.SemaphoreType.DMA] * NBUF,
        ],
        compiler_params=pltpu.CompilerParams(
            use_tc_tiling_on_sc=False, needs_layout_passes=False),
    )
    o5 = run(x_in, table)
    return (o5.reshape(S, D // 8, NW, 8, BB)
            .transpose(2, 4, 0, 1, 3).reshape(B, S, D))
